# Initial kernel scaffold; baseline (speedup 1.0000x reference)
import functools
from math import pi as PI

import jax
import jax.numpy as jnp
from jax import lax
from jax.experimental import pallas as pl
from jax.experimental.pallas import tpu as pltpu
from jax.experimental.pallas import tpu_sc as plsc

H = 128
NG = 50
TAB = 100
N = 10000
E = 320000
G = 256
CUT = 10.0
LOG2 = 0.6931471805599453

NC, NS, LANES = 2, 16, 16
NW = NC * NS
EPW = E // NW
N_PAD = 10240
RPS = N_PAD // NS
K = 80

EB = 2560
NBLK = E // EB

_sc_mesh = plsc.VectorSubcoreMesh(core_axis_name="c", subcore_axis_name="s")


def _ssp(x):
    return jnp.maximum(x, 0.0) + jnp.log(1.0 + jnp.exp(-jnp.abs(x))) - LOG2


def _edge_prep_body(src_h, dst_h, an_h, cx_h, cy_h, cz_h, a2_out, d2_out,
                    src_v, dst_v, an_v, cx_v, cy_v, cz_v, a2_v, d2_v):
    c = lax.axis_index("c")
    s = lax.axis_index("s")
    wid = s * NC + c
    base = wid * EPW
    pltpu.sync_copy(src_h.at[pl.ds(base, EPW)], src_v)
    pltpu.sync_copy(dst_h.at[pl.ds(base, EPW)], dst_v)
    pltpu.sync_copy(an_h, an_v)
    pltpu.sync_copy(cx_h, cx_v)
    pltpu.sync_copy(cy_h, cy_v)
    pltpu.sync_copy(cz_h, cz_v)

    @pl.loop(0, EPW // LANES)
    def _(i):
        sl = pl.ds(i * LANES, LANES)
        sv = src_v[sl]
        dv = dst_v[sl]
        a2 = plsc.load_gather(an_v, [sv])
        xs = plsc.load_gather(cx_v, [sv])
        xd = plsc.load_gather(cx_v, [dv])
        ys = plsc.load_gather(cy_v, [sv])
        yd = plsc.load_gather(cy_v, [dv])
        zs = plsc.load_gather(cz_v, [sv])
        zd = plsc.load_gather(cz_v, [dv])
        dx = xs - xd
        dy = ys - yd
        dz = zs - zd
        a2_v[sl] = a2
        d2_v[sl] = dx * dx + dy * dy + dz * dz

    pltpu.sync_copy(a2_v, a2_out.at[pl.ds(base, EPW)])
    pltpu.sync_copy(d2_v, d2_out.at[pl.ds(base, EPW)])


def _edge_prep(src, dst, atomic_ns, cx, cy, cz):
    return pl.kernel(
        _edge_prep_body,
        out_type=(jax.ShapeDtypeStruct((E,), jnp.int32),
                  jax.ShapeDtypeStruct((E,), jnp.float32)),
        mesh=_sc_mesh,
        scratch_types=[
            pltpu.VMEM((EPW,), jnp.int32),
            pltpu.VMEM((EPW,), jnp.int32),
            pltpu.VMEM((N,), jnp.int32),
            pltpu.VMEM((N,), jnp.float32),
            pltpu.VMEM((N,), jnp.float32),
            pltpu.VMEM((N,), jnp.float32),
            pltpu.VMEM((EPW,), jnp.int32),
            pltpu.VMEM((EPW,), jnp.float32),
        ],
    )(src, dst, atomic_ns, cx, cy, cz)


def _edge_mlp_body(a2_ref, d2_ref, emb_ref, nlw_ref, e1w_ref, e1b_ref,
                   e2w_ref, e2b_ref, out_ref, hw_ref):
    i = pl.program_id(0)

    @pl.when(i == 0)
    def _():
        hw_ref[...] = jnp.dot(emb_ref[...], nlw_ref[...],
                              preferred_element_type=jnp.float32)

    a2 = a2_ref[...]
    d2 = d2_ref[...]
    ew = jnp.sqrt(d2)
    step = CUT / (NG - 1)
    offs = lax.broadcasted_iota(jnp.float32, (1, NG), 1) * step
    coeff = -0.5 / step ** 2
    g = jnp.exp(coeff * (ew - offs) ** 2)
    t1 = _ssp(jnp.dot(g, e1w_ref[...], preferred_element_type=jnp.float32)
              + e1b_ref[...])
    w = jnp.dot(t1, e2w_ref[...], preferred_element_type=jnp.float32) \
        + e2b_ref[...]
    cutoff = 0.5 * (jnp.cos(ew * (PI / CUT)) + 1.0)
    cutoff = cutoff * (ew < CUT).astype(jnp.float32)
    onehot = (a2 == lax.broadcasted_iota(jnp.int32, (1, TAB), 1))
    hsrc = jnp.dot(onehot.astype(jnp.float32), hw_ref[...],
                   preferred_element_type=jnp.float32)
    out_ref[...] = hsrc * (w * cutoff)


def _edge_mlp(a2, d2, emb, nlw, e1w, e1b, e2w, e2b):
    return pl.pallas_call(
        _edge_mlp_body,
        grid=(NBLK,),
        in_specs=[
            pl.BlockSpec((EB, 1), lambda i: (i, 0)),
            pl.BlockSpec((EB, 1), lambda i: (i, 0)),
            pl.BlockSpec((TAB, H), lambda i: (0, 0)),
            pl.BlockSpec((H, H), lambda i: (0, 0)),
            pl.BlockSpec((NG, H), lambda i: (0, 0)),
            pl.BlockSpec((H,), lambda i: (0,)),
            pl.BlockSpec((H, H), lambda i: (0, 0)),
            pl.BlockSpec((H,), lambda i: (0,)),
        ],
        out_specs=pl.BlockSpec((EB, H), lambda i: (i, 0)),
        out_shape=jax.ShapeDtypeStruct((E, H), jnp.float32),
        scratch_shapes=[pltpu.VMEM((TAB, H), jnp.float32)],
        compiler_params=pltpu.CompilerParams(
            dimension_semantics=("arbitrary",)),
    )(a2, d2, emb, nlw, e1w, e1b, e2w, e2b)


def _scatter_body(dst_h, eo_h, z_h, out_h, idx_v, rows_v, agg_sh):
    c = lax.axis_index("c")
    s = lax.axis_index("s")
    wid = s * NC + c
    base = wid * EPW
    pltpu.sync_copy(z_h, agg_sh.at[pl.ds(s * RPS, RPS)])
    plsc.subcore_barrier()

    @pl.loop(0, EPW // K)
    def _(t):
        e0 = base + t * K
        pltpu.sync_copy(dst_h.at[pl.ds(e0, K)], idx_v.at[0])
        pltpu.sync_copy(eo_h.at[pl.ds(e0, K)], rows_v)
        pltpu.sync_copy(rows_v, agg_sh.at[idx_v.at[0]], add=True)

    plsc.subcore_barrier()
    pltpu.sync_copy(agg_sh.at[pl.ds(s * RPS, RPS)],
                    out_h.at[c].at[pl.ds(s * RPS, RPS)])


def _scatter_add(dst, edge_out):
    z = jnp.zeros((RPS, H), jnp.float32)
    return pl.kernel(
        _scatter_body,
        out_type=jax.ShapeDtypeStruct((NC, N_PAD, H), jnp.float32),
        mesh=_sc_mesh,
        scratch_types=[
            pltpu.VMEM((1, K), jnp.int32),
            pltpu.VMEM((K, H), jnp.float32),
            pltpu.VMEM_SHARED((N_PAD, H), jnp.float32),
        ],
    )(dst, edge_out, z)


def _node_post_body(aggp_ref, an_ref, bv_ref, emb_ref, n1w_ref, n1b_ref,
                    n2w_ref, n2b_ref, p1w_ref, p1b_ref, p2w_ref, p2b_ref,
                    node_out_ref, graph_ref):
    agg = aggp_ref[0, pl.ds(0, N), :] + aggp_ref[1, pl.ds(0, N), :]
    upd = jnp.dot(_ssp(jnp.dot(agg, n1w_ref[...],
                               preferred_element_type=jnp.float32)
                       + n1b_ref[...]),
                  n2w_ref[...], preferred_element_type=jnp.float32) \
        + n2b_ref[...]
    an = an_ref[...]
    onehot_a = (an == lax.broadcasted_iota(jnp.int32, (1, TAB), 1))
    node_embs = jnp.dot(onehot_a.astype(jnp.float32), emb_ref[...],
                        preferred_element_type=jnp.float32)
    x = node_embs + upd
    no = jnp.dot(_ssp(jnp.dot(x, p1w_ref[...],
                              preferred_element_type=jnp.float32)
                      + p1b_ref[...]),
                 p2w_ref[...], preferred_element_type=jnp.float32) \
        + p2b_ref[...]
    node_out_ref[...] = no
    bv = bv_ref[...]
    onehot_b = (bv == lax.broadcasted_iota(jnp.int32, (1, G), 1))
    graph_ref[...] = lax.dot_general(
        onehot_b.astype(jnp.float32), no,
        dimension_numbers=(((0,), (0,)), ((), ())),
        preferred_element_type=jnp.float32)


def _node_post(aggp, an, bv, emb, n1w, n1b, n2w, n2b, p1w, p1b, p2w, p2b):
    return pl.pallas_call(
        _node_post_body,
        out_shape=(jax.ShapeDtypeStruct((N, 1), jnp.float32),
                   jax.ShapeDtypeStruct((G, 1), jnp.float32)),
    )(aggp, an, bv, emb, n1w, n1b, n2w, n2b, p1w, p1b, p2w, p2b)


def kernel(atomic_ns, edge_index, coords, batch_node_vec, emb, node_lin_w,
           e1_w, e1_b, e2_w, e2_b, n1_w, n1_b, n2_w, n2_b,
           p1_w, p1_b, p2_w, p2_b):
    src = edge_index[0]
    dst = edge_index[1]
    cx = coords[:, 0]
    cy = coords[:, 1]
    cz = coords[:, 2]
    a2, d2 = _edge_prep(src, dst, atomic_ns.astype(jnp.int32), cx, cy, cz)
    edge_out = _edge_mlp(a2.reshape(E, 1), d2.reshape(E, 1), emb, node_lin_w,
                         e1_w, e1_b, e2_w, e2_b)
    aggp = _scatter_add(dst, edge_out)
    node_out, graph_emb = _node_post(
        aggp, atomic_ns.astype(jnp.int32).reshape(N, 1),
        batch_node_vec.astype(jnp.int32).reshape(N, 1), emb,
        n1_w, n1_b, n2_w, n2_b, p1_w, p1_b, p2_w, p2_b)
    return node_out, graph_emb


# speedup vs baseline: 2.6575x; 2.6575x over previous
import dataclasses
import functools
from math import pi as PI

import jax
import jax.numpy as jnp
from jax import lax
from jax.experimental import pallas as pl
from jax.experimental.pallas import tpu as pltpu
from jax.experimental.pallas import tpu_sc as plsc

H = 128
NG = 50
TAB = 100
N = 10000
E = 320000
G = 256
CUT = 10.0
LOG2 = 0.6931471805599453

NC, NS, LANES = 2, 16, 16
NW = NC * NS
EPW = E // NW
N_PAD = 10240
RPS = N_PAD // NS
K = 80

EB = 2560
NBLK = E // EB

def _sc_params():
    cp = pltpu.CompilerParams()
    if "needs_layout_passes" in pltpu.CompilerParams.__dataclass_fields__:
        cp = dataclasses.replace(cp, needs_layout_passes=False)
    return cp


def _sc_mesh():
    return plsc.VectorSubcoreMesh(core_axis_name="c", subcore_axis_name="s",
                                  num_cores=NC, num_subcores=NS)


def _ssp(x):
    return jnp.maximum(x, 0.0) + jnp.log(1.0 + jnp.exp(-jnp.abs(x))) - LOG2


def _edge_prep_body(src_h, dst_h, an_h, cx_h, cy_h, cz_h, a2_out, d2_out,
                    src_v, dst_v, an_v, cx_v, cy_v, cz_v, a2_v, d2_v):
    c = lax.axis_index("c")
    s = lax.axis_index("s")
    wid = s * NC + c
    base = wid * EPW
    pltpu.sync_copy(src_h.at[pl.ds(base, EPW)], src_v)
    pltpu.sync_copy(dst_h.at[pl.ds(base, EPW)], dst_v)
    pltpu.sync_copy(an_h, an_v)
    pltpu.sync_copy(cx_h, cx_v)
    pltpu.sync_copy(cy_h, cy_v)
    pltpu.sync_copy(cz_h, cz_v)

    @pl.loop(0, EPW // LANES)
    def _(i):
        sl = pl.ds(i * LANES, LANES)
        sv = src_v[sl]
        dv = dst_v[sl]
        a2 = plsc.load_gather(an_v, [sv])
        xs = plsc.load_gather(cx_v, [sv])
        xd = plsc.load_gather(cx_v, [dv])
        ys = plsc.load_gather(cy_v, [sv])
        yd = plsc.load_gather(cy_v, [dv])
        zs = plsc.load_gather(cz_v, [sv])
        zd = plsc.load_gather(cz_v, [dv])
        dx = xs - xd
        dy = ys - yd
        dz = zs - zd
        a2_v[sl] = a2
        d2_v[sl] = dx * dx + dy * dy + dz * dz

    pltpu.sync_copy(a2_v, a2_out.at[pl.ds(base, EPW)])
    pltpu.sync_copy(d2_v, d2_out.at[pl.ds(base, EPW)])


def _edge_prep(src, dst, atomic_ns, cx, cy, cz):
    return pl.kernel(
        _edge_prep_body,
        out_type=(jax.ShapeDtypeStruct((E,), jnp.int32),
                  jax.ShapeDtypeStruct((E,), jnp.float32)),
        mesh=_sc_mesh(),
        scratch_types=[
            pltpu.VMEM((EPW,), jnp.int32),
            pltpu.VMEM((EPW,), jnp.int32),
            pltpu.VMEM((N,), jnp.int32),
            pltpu.VMEM((N,), jnp.float32),
            pltpu.VMEM((N,), jnp.float32),
            pltpu.VMEM((N,), jnp.float32),
            pltpu.VMEM((EPW,), jnp.int32),
            pltpu.VMEM((EPW,), jnp.float32),
        ],
        compiler_params=_sc_params(),
    )(src, dst, atomic_ns, cx, cy, cz)


def _edge_mlp_body(a2_ref, d2_ref, emb_ref, nlw_ref, e1w_ref, e1b_ref,
                   e2w_ref, e2b_ref, out_ref, hw_ref):
    i = pl.program_id(0)

    @pl.when(i == 0)
    def _():
        hw_ref[...] = jnp.dot(emb_ref[...], nlw_ref[...],
                              preferred_element_type=jnp.float32)

    a2 = a2_ref[...]
    d2 = d2_ref[...]
    ew = jnp.sqrt(d2)
    step = CUT / (NG - 1)
    offs = lax.broadcasted_iota(jnp.int32, (1, NG), 1).astype(jnp.float32) * step
    coeff = -0.5 / step ** 2
    g = jnp.exp(coeff * (ew - offs) ** 2)
    t1 = _ssp(jnp.dot(g, e1w_ref[...], preferred_element_type=jnp.float32)
              + e1b_ref[...])
    w = jnp.dot(t1, e2w_ref[...], preferred_element_type=jnp.float32) \
        + e2b_ref[...]
    cutoff = 0.5 * (jnp.cos(ew * (PI / CUT)) + 1.0)
    cutoff = cutoff * (ew < CUT).astype(jnp.float32)
    onehot = (a2 == lax.broadcasted_iota(jnp.int32, (1, TAB), 1))
    hsrc = jnp.dot(onehot.astype(jnp.float32), hw_ref[...],
                   preferred_element_type=jnp.float32)
    out_ref[...] = hsrc * (w * cutoff)


def _edge_mlp(a2, d2, emb, nlw, e1w, e1b, e2w, e2b):
    return pl.pallas_call(
        _edge_mlp_body,
        grid=(NBLK,),
        in_specs=[
            pl.BlockSpec((EB, 1), lambda i: (i, 0)),
            pl.BlockSpec((EB, 1), lambda i: (i, 0)),
            pl.BlockSpec((TAB, H), lambda i: (0, 0)),
            pl.BlockSpec((H, H), lambda i: (0, 0)),
            pl.BlockSpec((NG, H), lambda i: (0, 0)),
            pl.BlockSpec((H,), lambda i: (0,)),
            pl.BlockSpec((H, H), lambda i: (0, 0)),
            pl.BlockSpec((H,), lambda i: (0,)),
        ],
        out_specs=pl.BlockSpec((EB, H), lambda i: (i, 0)),
        out_shape=jax.ShapeDtypeStruct((E, H), jnp.float32),
        scratch_shapes=[pltpu.VMEM((TAB, H), jnp.float32)],
        compiler_params=pltpu.CompilerParams(
            dimension_semantics=("arbitrary",)),
    )(a2, d2, emb, nlw, e1w, e1b, e2w, e2b)


def _scatter_body(dst_h, eo_h, z_h, out_h, idx_v, rows_v, agg_sh):
    c = lax.axis_index("c")
    s = lax.axis_index("s")
    wid = s * NC + c
    base = wid * EPW
    pltpu.sync_copy(z_h, agg_sh.at[pl.ds(s * RPS, RPS)])
    plsc.subcore_barrier()

    @pl.loop(0, EPW // K)
    def _(t):
        e0 = base + t * K
        pltpu.sync_copy(dst_h.at[pl.ds(e0, K)], idx_v.at[0])
        pltpu.sync_copy(eo_h.at[pl.ds(e0, K)], rows_v)
        pltpu.sync_copy(rows_v, agg_sh.at[idx_v.at[0]], add=True)

    plsc.subcore_barrier()
    pltpu.sync_copy(agg_sh.at[pl.ds(s * RPS, RPS)],
                    out_h.at[c].at[pl.ds(s * RPS, RPS)])


def _scatter_add(dst, edge_out):
    z = jnp.zeros((RPS, H), jnp.float32)
    return pl.kernel(
        _scatter_body,
        out_type=jax.ShapeDtypeStruct((NC, N_PAD, H), jnp.float32),
        mesh=_sc_mesh(),
        scratch_types=[
            pltpu.VMEM((1, K), jnp.int32),
            pltpu.VMEM((K, H), jnp.float32),
            pltpu.VMEM_SHARED((N_PAD, H), jnp.float32),
        ],
    )(dst, edge_out, z)


def _node_post_body(aggp_ref, an_ref, bv_ref, emb_ref, n1w_ref, n1b_ref,
                    n2w_ref, n2b_ref, p1w_ref, p1b_ref, p2w_ref, p2b_ref,
                    node_out_ref, graph_ref):
    agg = aggp_ref[0, pl.ds(0, N), :] + aggp_ref[1, pl.ds(0, N), :]
    upd = jnp.dot(_ssp(jnp.dot(agg, n1w_ref[...],
                               preferred_element_type=jnp.float32)
                       + n1b_ref[...]),
                  n2w_ref[...], preferred_element_type=jnp.float32) \
        + n2b_ref[...]
    an = an_ref[...]
    onehot_a = (an == lax.broadcasted_iota(jnp.int32, (1, TAB), 1))
    node_embs = jnp.dot(onehot_a.astype(jnp.float32), emb_ref[...],
                        preferred_element_type=jnp.float32)
    x = node_embs + upd
    no = jnp.dot(_ssp(jnp.dot(x, p1w_ref[...],
                              preferred_element_type=jnp.float32)
                      + p1b_ref[...]),
                 p2w_ref[...], preferred_element_type=jnp.float32) \
        + p2b_ref[...]
    node_out_ref[...] = no
    bv = bv_ref[...]
    onehot_b = (bv == lax.broadcasted_iota(jnp.int32, (1, G), 1))
    graph_ref[...] = lax.dot_general(
        onehot_b.astype(jnp.float32), no,
        dimension_numbers=(((0,), (0,)), ((), ())),
        preferred_element_type=jnp.float32)


def _node_post(aggp, an, bv, emb, n1w, n1b, n2w, n2b, p1w, p1b, p2w, p2b):
    return pl.pallas_call(
        _node_post_body,
        out_shape=(jax.ShapeDtypeStruct((N, 1), jnp.float32),
                   jax.ShapeDtypeStruct((G, 1), jnp.float32)),
    )(aggp, an, bv, emb, n1w, n1b, n2w, n2b, p1w, p1b, p2w, p2b)


def kernel(atomic_ns, edge_index, coords, batch_node_vec, emb, node_lin_w,
           e1_w, e1_b, e2_w, e2_b, n1_w, n1_b, n2_w, n2_b,
           p1_w, p1_b, p2_w, p2_b):
    src = edge_index[0]
    dst = edge_index[1]
    cx = coords[:, 0]
    cy = coords[:, 1]
    cz = coords[:, 2]
    a2, d2 = _edge_prep(src, dst, atomic_ns.astype(jnp.int32), cx, cy, cz)
    edge_out = _edge_mlp(a2.reshape(E, 1), d2.reshape(E, 1), emb, node_lin_w,
                         e1_w, e1_b, e2_w, e2_b)
    aggp = _scatter_add(dst, edge_out)
    node_out, graph_emb = _node_post(
        aggp, atomic_ns.astype(jnp.int32).reshape(N, 1),
        batch_node_vec.astype(jnp.int32).reshape(N, 1), emb,
        n1_w, n1_b, n2_w, n2_b, p1_w, p1_b, p2_w, p2_b)
    return node_out, graph_emb


# speedup vs baseline: 3.5507x; 1.3361x over previous
import dataclasses
import functools
from math import pi as PI

import jax
import jax.numpy as jnp
from jax import lax
from jax.experimental import pallas as pl
from jax.experimental.pallas import tpu as pltpu
from jax.experimental.pallas import tpu_sc as plsc

H = 128
NG = 50
TAB = 100
N = 10000
E = 320000
G = 256
CUT = 10.0
LOG2 = 0.6931471805599453

NC, NS, LANES = 2, 16, 16
NW = NC * NS
EPW = E // NW
N_PAD = 10240
RPS = N_PAD // NS
K = 80

EB = 2560
NBLK = E // EB

def _sc_params():
    cp = pltpu.CompilerParams()
    if "needs_layout_passes" in pltpu.CompilerParams.__dataclass_fields__:
        cp = dataclasses.replace(cp, needs_layout_passes=False)
    return cp


def _sc_mesh():
    return plsc.VectorSubcoreMesh(core_axis_name="c", subcore_axis_name="s",
                                  num_cores=NC, num_subcores=NS)


def _ssp(x):
    return jnp.maximum(x, 0.0) + jnp.log(1.0 + jnp.exp(-jnp.abs(x))) - LOG2


def _edge_prep_body(src_h, dst_h, an_h, cx_h, cy_h, cz_h, a2_out, d2_out,
                    src_v, dst_v, an_v, cx_v, cy_v, cz_v, a2_v, d2_v):
    c = lax.axis_index("c")
    s = lax.axis_index("s")
    wid = s * NC + c
    base = wid * EPW
    pltpu.sync_copy(src_h.at[pl.ds(base, EPW)], src_v)
    pltpu.sync_copy(dst_h.at[pl.ds(base, EPW)], dst_v)
    pltpu.sync_copy(an_h, an_v)
    pltpu.sync_copy(cx_h, cx_v)
    pltpu.sync_copy(cy_h, cy_v)
    pltpu.sync_copy(cz_h, cz_v)

    @pl.loop(0, EPW // LANES)
    def _(i):
        sl = pl.ds(i * LANES, LANES)
        sv = src_v[sl]
        dv = dst_v[sl]
        a2 = plsc.load_gather(an_v, [sv])
        xs = plsc.load_gather(cx_v, [sv])
        xd = plsc.load_gather(cx_v, [dv])
        ys = plsc.load_gather(cy_v, [sv])
        yd = plsc.load_gather(cy_v, [dv])
        zs = plsc.load_gather(cz_v, [sv])
        zd = plsc.load_gather(cz_v, [dv])
        dx = xs - xd
        dy = ys - yd
        dz = zs - zd
        a2_v[sl] = a2
        d2_v[sl] = dx * dx + dy * dy + dz * dz

    pltpu.sync_copy(a2_v, a2_out.at[pl.ds(base, EPW)])
    pltpu.sync_copy(d2_v, d2_out.at[pl.ds(base, EPW)])


def _edge_prep(src, dst, atomic_ns, cx, cy, cz):
    return pl.kernel(
        _edge_prep_body,
        out_type=(jax.ShapeDtypeStruct((E,), jnp.int32),
                  jax.ShapeDtypeStruct((E,), jnp.float32)),
        mesh=_sc_mesh(),
        scratch_types=[
            pltpu.VMEM((EPW,), jnp.int32),
            pltpu.VMEM((EPW,), jnp.int32),
            pltpu.VMEM((N,), jnp.int32),
            pltpu.VMEM((N,), jnp.float32),
            pltpu.VMEM((N,), jnp.float32),
            pltpu.VMEM((N,), jnp.float32),
            pltpu.VMEM((EPW,), jnp.int32),
            pltpu.VMEM((EPW,), jnp.float32),
        ],
        compiler_params=_sc_params(),
    )(src, dst, atomic_ns, cx, cy, cz)


def _scalar_prep_body(d2_ref, ew_ref, cut_ref):
    d2 = d2_ref[...]
    ew = jnp.sqrt(d2)
    cut = 0.5 * (jnp.cos(ew * (PI / CUT)) + 1.0)
    cut_ref[...] = cut * (ew < CUT).astype(jnp.float32)
    ew_ref[...] = ew


def _scalar_prep(d2):
    return pl.pallas_call(
        _scalar_prep_body,
        out_shape=(jax.ShapeDtypeStruct((E // 320, 320), jnp.float32),
                   jax.ShapeDtypeStruct((E // 320, 320), jnp.float32)),
    )(d2.reshape(E // 320, 320))


def _edge_mlp_body(a2_ref, ew_ref, cut_ref, emb_ref, nlw_ref, e1w_ref,
                   e1b_ref, e2w_ref, e2b_ref, out_ref, hw_ref):
    i = pl.program_id(0)

    @pl.when(i == 0)
    def _():
        hw_ref[...] = jnp.dot(emb_ref[...], nlw_ref[...],
                              preferred_element_type=jnp.float32)

    a2 = a2_ref[...]
    ew = ew_ref[...]
    cutoff = cut_ref[...]
    step = CUT / (NG - 1)
    offs = lax.broadcasted_iota(jnp.int32, (1, NG), 1).astype(jnp.float32) * step
    coeff = -0.5 / step ** 2
    g = jnp.exp(coeff * (ew - offs) ** 2)
    t1 = _ssp(jnp.dot(g, e1w_ref[...], preferred_element_type=jnp.float32)
              + e1b_ref[...])
    w = jnp.dot(t1, e2w_ref[...], preferred_element_type=jnp.float32) \
        + e2b_ref[...]
    onehot = (a2 == lax.broadcasted_iota(jnp.int32, (1, TAB), 1))
    hsrc = jnp.dot(onehot.astype(jnp.float32), hw_ref[...],
                   preferred_element_type=jnp.float32)
    out_ref[...] = hsrc * (w * cutoff)


def _edge_mlp(a2, ew, cut, emb, nlw, e1w, e1b, e2w, e2b):
    return pl.pallas_call(
        _edge_mlp_body,
        grid=(NBLK,),
        in_specs=[
            pl.BlockSpec((EB, 1), lambda i: (i, 0)),
            pl.BlockSpec((EB, 1), lambda i: (i, 0)),
            pl.BlockSpec((EB, 1), lambda i: (i, 0)),
            pl.BlockSpec((TAB, H), lambda i: (0, 0)),
            pl.BlockSpec((H, H), lambda i: (0, 0)),
            pl.BlockSpec((NG, H), lambda i: (0, 0)),
            pl.BlockSpec((H,), lambda i: (0,)),
            pl.BlockSpec((H, H), lambda i: (0, 0)),
            pl.BlockSpec((H,), lambda i: (0,)),
        ],
        out_specs=pl.BlockSpec((EB, H), lambda i: (i, 0)),
        out_shape=jax.ShapeDtypeStruct((E, H), jnp.float32),
        scratch_shapes=[pltpu.VMEM((TAB, H), jnp.float32)],
        compiler_params=pltpu.CompilerParams(
            dimension_semantics=("arbitrary",)),
    )(a2, ew, cut, emb, nlw, e1w, e1b, e2w, e2b)


NCH = EPW // K


def _scatter_body(dst_h, eo_h, z_h, out_h, idx_v, rows_v, agg_sh, sems):
    c = lax.axis_index("c")
    s = lax.axis_index("s")
    wid = s * NC + c
    base = wid * EPW
    pltpu.sync_copy(z_h, agg_sh.at[pl.ds(s * RPS, RPS)])
    pltpu.sync_copy(dst_h.at[wid], idx_v)
    plsc.subcore_barrier()

    def _start(t, b):
        pltpu.make_async_copy(eo_h.at[pl.ds(base + t * K, K)],
                              rows_v.at[b], sems.at[b]).start()

    def _wait(t, b):
        pltpu.make_async_copy(eo_h.at[pl.ds(base + t * K, K)],
                              rows_v.at[b], sems.at[b]).wait()

    def _scat(t, b):
        pltpu.sync_copy(rows_v.at[b], agg_sh.at[idx_v.at[t]], add=True)

    _start(0, 0)

    @pl.loop(0, (NCH - 1) // 2)
    def _(u):
        t0 = 2 * u
        _start(t0 + 1, 1)
        _wait(t0, 0)
        _scat(t0, 0)
        _start(t0 + 2, 0)
        _wait(t0 + 1, 1)
        _scat(t0 + 1, 1)

    _wait(NCH - 1, 0)
    _scat(NCH - 1, 0)

    plsc.subcore_barrier()
    pltpu.sync_copy(agg_sh.at[pl.ds(s * RPS, RPS)],
                    out_h.at[c].at[pl.ds(s * RPS, RPS)])


def _scatter_add(dst, edge_out):
    z = jnp.zeros((RPS, H), jnp.float32)
    return pl.kernel(
        _scatter_body,
        out_type=jax.ShapeDtypeStruct((NC, N_PAD, H), jnp.float32),
        mesh=_sc_mesh(),
        scratch_types=[
            pltpu.VMEM((NCH, K), jnp.int32),
            pltpu.VMEM((2, K, H), jnp.float32),
            pltpu.VMEM_SHARED((N_PAD, H), jnp.float32),
            pltpu.SemaphoreType.DMA((2,)),
        ],
        compiler_params=_sc_params(),
    )(dst.reshape(NW, NCH, K), edge_out, z)


def _node_post_body(aggp_ref, an_ref, bv_ref, emb_ref, n1w_ref, n1b_ref,
                    n2w_ref, n2b_ref, p1w_ref, p1b_ref, p2w_ref, p2b_ref,
                    node_out_ref, graph_ref):
    agg = aggp_ref[0, pl.ds(0, N), :] + aggp_ref[1, pl.ds(0, N), :]
    upd = jnp.dot(_ssp(jnp.dot(agg, n1w_ref[...],
                               preferred_element_type=jnp.float32)
                       + n1b_ref[...]),
                  n2w_ref[...], preferred_element_type=jnp.float32) \
        + n2b_ref[...]
    an = an_ref[...]
    onehot_a = (an == lax.broadcasted_iota(jnp.int32, (1, TAB), 1))
    node_embs = jnp.dot(onehot_a.astype(jnp.float32), emb_ref[...],
                        preferred_element_type=jnp.float32)
    x = node_embs + upd
    no = jnp.dot(_ssp(jnp.dot(x, p1w_ref[...],
                              preferred_element_type=jnp.float32)
                      + p1b_ref[...]),
                 p2w_ref[...], preferred_element_type=jnp.float32) \
        + p2b_ref[...]
    node_out_ref[...] = no
    bv = bv_ref[...]
    onehot_b = (bv == lax.broadcasted_iota(jnp.int32, (1, G), 1))
    graph_ref[...] = lax.dot_general(
        onehot_b.astype(jnp.float32), no,
        dimension_numbers=(((0,), (0,)), ((), ())),
        preferred_element_type=jnp.float32)


def _node_post(aggp, an, bv, emb, n1w, n1b, n2w, n2b, p1w, p1b, p2w, p2b):
    return pl.pallas_call(
        _node_post_body,
        out_shape=(jax.ShapeDtypeStruct((N, 1), jnp.float32),
                   jax.ShapeDtypeStruct((G, 1), jnp.float32)),
    )(aggp, an, bv, emb, n1w, n1b, n2w, n2b, p1w, p1b, p2w, p2b)


def kernel(atomic_ns, edge_index, coords, batch_node_vec, emb, node_lin_w,
           e1_w, e1_b, e2_w, e2_b, n1_w, n1_b, n2_w, n2_b,
           p1_w, p1_b, p2_w, p2_b):
    src = edge_index[0]
    dst = edge_index[1]
    cx = coords[:, 0]
    cy = coords[:, 1]
    cz = coords[:, 2]
    a2, d2 = _edge_prep(src, dst, atomic_ns.astype(jnp.int32), cx, cy, cz)
    ew, cut = _scalar_prep(d2)
    edge_out = _edge_mlp(a2.reshape(E, 1), ew.reshape(E, 1), cut.reshape(E, 1),
                         emb, node_lin_w, e1_w, e1_b, e2_w, e2_b)
    aggp = _scatter_add(dst, edge_out)
    node_out, graph_emb = _node_post(
        aggp, atomic_ns.astype(jnp.int32).reshape(N, 1),
        batch_node_vec.astype(jnp.int32).reshape(N, 1), emb,
        n1_w, n1_b, n2_w, n2_b, p1_w, p1_b, p2_w, p2_b)
    return node_out, graph_emb
